# Initial kernel scaffold; baseline (speedup 1.0000x reference)
import jax
import jax.numpy as jnp
from jax.experimental import pallas as pl


def kernel(input_ids, embed_table, W, b):
    raise NotImplementedError("write your pallas kernel here")


# speedup vs baseline: 1.5942x; 1.5942x over previous
import functools

import jax
import jax.numpy as jnp
from jax import lax
from jax.experimental import pallas as pl
from jax.experimental.pallas import tpu as pltpu
from jax.experimental.pallas import tpu_sc as plsc

VOCAB = 1000000
HIDDEN = 32
BATCH = 4096
SEQ = 200


ROWS_PER_BLOCK = 8000
NUM_BLOCKS = VOCAB // ROWS_PER_BLOCK


def _matvec_body(x_ref, wt_ref, b_ref, t_ref):
    x = x_ref[...]
    t_ref[...] = jnp.sum(x * wt_ref[...], axis=1, keepdims=True) + b_ref[...]


def _compute_t(embed_table, wt, b2):
    return pl.pallas_call(
        _matvec_body,
        grid=(NUM_BLOCKS,),
        in_specs=[
            pl.BlockSpec((ROWS_PER_BLOCK, HIDDEN), lambda i: (i, 0)),
            pl.BlockSpec((1, HIDDEN), lambda i: (0, 0)),
            pl.BlockSpec((1, 1), lambda i: (0, 0)),
        ],
        out_specs=pl.BlockSpec((ROWS_PER_BLOCK, 1), lambda i: (i, 0)),
        out_shape=jax.ShapeDtypeStruct((VOCAB, 1), jnp.float32),
    )(embed_table, wt, b2)


NUM_WORKERS = 32
TOK_PER_W = BATCH * SEQ // NUM_WORKERS
SAMP_PER_W = BATCH // NUM_WORKERS
CHUNK = 128
CHUNKS_PER_W = TOK_PER_W // CHUNK
FIRE = 8


def _pool_body(ids_hbm, t_hbm, out_hbm, idx_v, vals_v, out_v, sem):
    wid = lax.axis_index("s") * 2 + lax.axis_index("c")
    row0 = wid * CHUNKS_PER_W

    pltpu.sync_copy(ids_hbm.at[pl.ds(row0, CHUNKS_PER_W)], idx_v)

    def fire_drain(outer, _):
        cps = []
        for j in range(FIRE):
            r = outer * FIRE + j
            cps.append(pltpu.async_copy(
                t_hbm.at[idx_v.at[r]], vals_v.at[pl.ds(r * CHUNK, CHUNK)], sem))
        for cp in cps:
            cp.wait()
        return 0

    lax.fori_loop(0, CHUNKS_PER_W // FIRE, fire_drain, 0)

    lane = lax.iota(jnp.int32, 16)
    for g in range(SAMP_PER_W // 16):
        base_f = (g * 16 + lane) * SEQ

        def body(j, acc):
            v = plsc.load_gather(vals_v, [base_f + j])
            return acc + v

        acc = lax.fori_loop(0, SEQ, body, jnp.zeros((16,), jnp.float32))
        out_v[pl.ds(g * 16, 16)] = acc

    pltpu.sync_copy(out_v, out_hbm.at[pl.ds(wid * SAMP_PER_W, SAMP_PER_W)])


@functools.lru_cache(maxsize=1)
def _make_pool():
    return functools.partial(
        pl.kernel,
        mesh=plsc.VectorSubcoreMesh(core_axis_name="c", subcore_axis_name="s"),
        compiler_params=pltpu.CompilerParams(needs_layout_passes=False),
        out_type=jax.ShapeDtypeStruct((BATCH,), jnp.float32),
        scratch_types=[
            pltpu.VMEM((CHUNKS_PER_W, CHUNK), jnp.int32),
            pltpu.VMEM((TOK_PER_W,), jnp.float32),
            pltpu.VMEM((SAMP_PER_W,), jnp.float32),
            pltpu.SemaphoreType.DMA,
        ],
    )(_pool_body)


def kernel(input_ids, embed_table, W, b):
    ids = input_ids.astype(jnp.int32).reshape(BATCH * SEQ // CHUNK, CHUNK)
    wt = (W.astype(jnp.float32) / SEQ).reshape(1, HIDDEN)
    b2 = (b.astype(jnp.float32) / SEQ).reshape(1, 1)
    t = _compute_t(embed_table, wt, b2).reshape(VOCAB)
    pooled = _make_pool()(ids, t)
    return pooled.reshape(BATCH, 1)


# speedup vs baseline: 1.6270x; 1.0206x over previous
import functools

import jax
import jax.numpy as jnp
from jax import lax
from jax.experimental import pallas as pl
from jax.experimental.pallas import tpu as pltpu
from jax.experimental.pallas import tpu_sc as plsc

VOCAB = 1000000
HIDDEN = 32
BATCH = 4096
SEQ = 200


BLK_Y = 2000
VPB = 4 * BLK_Y
NUM_BLOCKS = VOCAB // VPB
GROUP = 32
VPG = GROUP * VPB
NUM_GROUPS = -(-NUM_BLOCKS // GROUP)
T_ROWS = NUM_GROUPS * BLK_Y
T_SIZE = T_ROWS * 128


def _matvec_body(y_ref, s_ref, b_ref, o_ref):
    g = pl.program_id(0)
    j = g % GROUP
    z = jnp.dot(y_ref[...], s_ref[...], preferred_element_type=jnp.float32)
    contrib = pltpu.roll(z, 4 * j, axis=1)

    @pl.when(j == 0)
    def _():
        o_ref[...] = contrib + b_ref[...]

    @pl.when(j != 0)
    def _():
        o_ref[...] += contrib


def _compute_t(table_view, s_mat, b2):
    return pl.pallas_call(
        _matvec_body,
        grid=(NUM_BLOCKS,),
        in_specs=[
            pl.BlockSpec((BLK_Y, 128), lambda i: (i, 0)),
            pl.BlockSpec((128, 128), lambda i: (0, 0)),
            pl.BlockSpec((1, 1), lambda i: (0, 0)),
        ],
        out_specs=pl.BlockSpec((BLK_Y, 128), lambda i: (i // GROUP, 0)),
        out_shape=jax.ShapeDtypeStruct((T_ROWS, 128), jnp.float32),
    )(table_view, s_mat, b2)


NUM_WORKERS = 32
TOK_PER_W = BATCH * SEQ // NUM_WORKERS
SAMP_PER_W = BATCH // NUM_WORKERS
ROWS_PER_W = TOK_PER_W // 128


def _pool_body(ids_hbm, t_hbm, out_hbm, idx_v, vals_v, out_v, sem):
    wid = lax.axis_index("s") * 2 + lax.axis_index("c")
    row0 = wid * ROWS_PER_W

    pltpu.sync_copy(ids_hbm.at[pl.ds(row0, ROWS_PER_W)], idx_v)

    def xform(i, carry):
        for u in range(128 // 16):
            v = idx_v[i, pl.ds(u * 16, 16)]
            big_g = v // VPG
            m = v - big_g * VPG
            jb = m // VPB
            q = m - jb * VPB
            idx_v[i, pl.ds(u * 16, 16)] = (
                big_g * VPG
                + lax.shift_left(lax.shift_right_logical(q, 2), 7)
                + lax.shift_left(jb, 2)
                + lax.bitwise_and(q, 3)
            )
        return carry

    lax.fori_loop(0, ROWS_PER_W, xform, 0)

    FIRE = 8

    def fire_drain(outer, carry):
        cps = []
        for j in range(FIRE):
            r = outer * FIRE + j
            cps.append(pltpu.async_copy(t_hbm.at[idx_v.at[r]], vals_v.at[r], sem))
        for cp in cps:
            cp.wait()
        return carry

    lax.fori_loop(0, ROWS_PER_W // FIRE, fire_drain, 0)

    lane = lax.iota(jnp.int32, 16)
    for g in range(SAMP_PER_W // 16):
        base_f = (g * 16 + lane) * SEQ

        def body(j, acc):
            f = base_f + j
            v = plsc.load_gather(vals_v, [lax.shift_right_logical(f, 7),
                                          lax.bitwise_and(f, 127)])
            return acc + v

        acc = lax.fori_loop(0, SEQ, body, jnp.zeros((16,), jnp.float32))
        out_v[pl.ds(g * 16, 16)] = acc

    pltpu.sync_copy(out_v, out_hbm.at[pl.ds(wid * SAMP_PER_W, SAMP_PER_W)])


@functools.lru_cache(maxsize=1)
def _make_pool():
    return functools.partial(
        pl.kernel,
        mesh=plsc.VectorSubcoreMesh(core_axis_name="c", subcore_axis_name="s"),
        compiler_params=pltpu.CompilerParams(needs_layout_passes=False),
        out_type=jax.ShapeDtypeStruct((BATCH,), jnp.float32),
        scratch_types=[
            pltpu.VMEM((ROWS_PER_W, 128), jnp.int32),
            pltpu.VMEM((ROWS_PER_W, 128), jnp.float32),
            pltpu.VMEM((SAMP_PER_W,), jnp.float32),
            pltpu.SemaphoreType.DMA,
        ],
    )(_pool_body)


def kernel(input_ids, embed_table, W, b):
    ids = input_ids.astype(jnp.int32).reshape(BATCH * SEQ // 128, 128)
    table_view = embed_table.reshape(VOCAB // 4, 128)
    k = jnp.arange(128)
    wtile = jnp.tile(W.reshape(HIDDEN).astype(jnp.float32), 4) / SEQ
    s_mat = jnp.where(jnp.arange(128)[None, :] == (k[:, None] // HIDDEN),
                      wtile[:, None], 0.0).astype(jnp.float32)
    b2 = (b.astype(jnp.float32) / SEQ).reshape(1, 1)
    t = _compute_t(table_view, s_mat, b2).reshape(T_SIZE)
    pooled = _make_pool()(ids, t)
    return pooled.reshape(BATCH, 1)
